# Initial kernel scaffold; baseline (speedup 1.0000x reference)
import jax
import jax.numpy as jnp
from jax.experimental import pallas as pl


def kernel(vectors, node_feats, radial_embedding, senders, receivers, W_up, M1, M2, M3, M4, Wd0, Wd1):
    raise NotImplementedError("write your pallas kernel here")


# speedup vs baseline: 2.0024x; 2.0024x over previous
import functools
import math

import jax
import jax.numpy as jnp
from jax import lax
from jax.experimental import pallas as pl
from jax.experimental.pallas import tpu as pltpu
from jax.experimental.pallas import tpu_sc as plsc

N = 10000
E = 160000
C = 64
RAD = 8
AVG_NEIGH = 16.0

NC = 2
NS = 16
K = 80
EPC = E // NS
NCH = EPC // K
DR = 80
NDR_TOT = N // DR
NDR_MAX = (NDR_TOT + NS - 1) // NS

_SH_COEF = math.sqrt(3.0 / (4.0 * math.pi))


def _h_body(nf_ref, w_ref, h_ref):
    h_ref[...] = jnp.dot(nf_ref[...], w_ref[...],
                         preferred_element_type=jnp.float32)


def _compute_h(node_feats, w_up_s):
    bn = 2000
    return pl.pallas_call(
        _h_body,
        grid=(N // bn,),
        in_specs=[
            pl.BlockSpec((bn, C), lambda i: (i, 0)),
            pl.BlockSpec((C, C), lambda i: (0, 0)),
        ],
        out_specs=pl.BlockSpec((bn, C), lambda i: (i, 0)),
        out_shape=jax.ShapeDtypeStruct((N, C), jnp.float32),
    )(node_feats, w_up_s)


def _edge_body(rad_ref, vx_ref, vy_ref, vz_ref, m1_ref, m2_ref, m3_ref,
               m4_ref, mix_ref, s0_ref, s1_ref, s2_ref):
    x = jnp.dot(rad_ref[...], m1_ref[...], preferred_element_type=jnp.float32)
    x = x * lax.logistic(x)
    x = jnp.dot(x, m2_ref[...], preferred_element_type=jnp.float32)
    x = x * lax.logistic(x)
    x = jnp.dot(x, m3_ref[...], preferred_element_type=jnp.float32)
    x = x * lax.logistic(x)
    mix_ref[...] = jnp.dot(x, m4_ref[...], preferred_element_type=jnp.float32)

    vx, vy, vz = vx_ref[...], vy_ref[...], vz_ref[...]
    inv = lax.rsqrt(vx * vx + vy * vy + vz * vz + 1e-12) * _SH_COEF
    s0_ref[...] = vx * inv
    s1_ref[...] = vy * inv
    s2_ref[...] = vz * inv


def _edge_prep(radial, vx, vy, vz, m1s, m2s, m3s, m4s):
    be = 2048
    grid = (pl.cdiv(E, be),)
    return pl.pallas_call(
        _edge_body,
        grid=grid,
        in_specs=[
            pl.BlockSpec((be, RAD), lambda i: (i, 0)),
            pl.BlockSpec((be,), lambda i: (i,)),
            pl.BlockSpec((be,), lambda i: (i,)),
            pl.BlockSpec((be,), lambda i: (i,)),
            pl.BlockSpec((RAD, 64), lambda i: (0, 0)),
            pl.BlockSpec((64, 64), lambda i: (0, 0)),
            pl.BlockSpec((64, 64), lambda i: (0, 0)),
            pl.BlockSpec((64, 2 * C), lambda i: (0, 0)),
        ],
        out_specs=[
            pl.BlockSpec((be, 2 * C), lambda i: (i, 0)),
            pl.BlockSpec((be,), lambda i: (i,)),
            pl.BlockSpec((be,), lambda i: (i,)),
            pl.BlockSpec((be,), lambda i: (i,)),
        ],
        out_shape=[
            jax.ShapeDtypeStruct((E, 2 * C), jnp.float32),
            jax.ShapeDtypeStruct((E,), jnp.float32),
            jax.ShapeDtypeStruct((E,), jnp.float32),
            jax.ShapeDtypeStruct((E,), jnp.float32),
        ],
    )(radial, vx, vy, vz, m1s, m2s, m3s, m4s)


def _sc_body(h_hbm, mix_hbm, s0_hbm, s1_hbm, s2_hbm, send_hbm, recv_hbm,
             out_hbm, acc_sh, zbuf, idx_s, idx_r, u_v, mix_v, s0_v, s1_v,
             s2_v, msg_v, sem):
    cid = lax.axis_index("c")
    sid = lax.axis_index("s")

    def _zrow(i, _):
        for q in range(8):
            zbuf[i, pl.ds(16 * q, 16)] = jnp.zeros((16,), jnp.float32)
        return 0
    lax.fori_loop(0, DR, _zrow, 0)
    for kdr in range(NDR_MAX):
        ch = sid + NS * kdr

        @pl.when(ch < NDR_TOT)
        def _():
            pltpu.sync_copy(zbuf, acc_sh.at[pl.ds(ch * DR, DR), :])
    plsc.subcore_barrier()

    ebase = sid * EPC

    def _chunk(ci, _):
        base = ebase + ci * K
        pltpu.sync_copy(send_hbm.at[pl.ds(base, K)], idx_s)
        gather = pltpu.async_copy(h_hbm.at[idx_s], u_v, sem)
        pltpu.sync_copy(recv_hbm.at[pl.ds(base, K)], idx_r)

        pltpu.sync_copy(mix_hbm.at[pl.ds(base, K), :], mix_v)

        @pl.when(cid == 0)
        def _():
            pltpu.sync_copy(s0_hbm.at[pl.ds(base, K)], s0_v)

        @pl.when(cid == 1)
        def _():
            pltpu.sync_copy(s1_hbm.at[pl.ds(base, K)], s1_v)
            pltpu.sync_copy(s2_hbm.at[pl.ds(base, K)], s2_v)

        gather.wait()

        @pl.when(cid == 0)
        def _():
            def gb(g, _):
                s0vec = s0_v[pl.ds(16 * g, 16)]
                for lane in range(16):
                    j = 16 * g + lane
                    s0 = s0vec[lane]
                    for q in range(4):
                        uq = u_v[j, pl.ds(16 * q, 16)]
                        m0q = mix_v[j, pl.ds(16 * q, 16)]
                        m1q = mix_v[j, pl.ds(64 + 16 * q, 16)]
                        msg_v[j, pl.ds(16 * q, 16)] = uq * m0q
                        msg_v[j, pl.ds(64 + 16 * q, 16)] = uq * m1q * s0
                return 0
            lax.fori_loop(0, K // 16, gb, 0)

        @pl.when(cid == 1)
        def _():
            def gb(g, _):
                s1vec = s1_v[pl.ds(16 * g, 16)]
                s2vec = s2_v[pl.ds(16 * g, 16)]
                for lane in range(16):
                    j = 16 * g + lane
                    s1 = s1vec[lane]
                    s2 = s2vec[lane]
                    for q in range(4):
                        uq = u_v[j, pl.ds(16 * q, 16)]
                        m1q = mix_v[j, pl.ds(64 + 16 * q, 16)]
                        t = uq * m1q
                        msg_v[j, pl.ds(16 * q, 16)] = t * s1
                        msg_v[j, pl.ds(64 + 16 * q, 16)] = t * s2
                return 0
            lax.fori_loop(0, K // 16, gb, 0)

        pltpu.sync_copy(msg_v, acc_sh.at[idx_r], add=True)
        return 0

    lax.fori_loop(0, NCH, _chunk, 0)

    plsc.subcore_barrier()
    for kdr in range(NDR_MAX):
        ch = sid + NS * kdr

        @pl.when(ch < NDR_TOT)
        def _():
            r0 = ch * DR
            pltpu.sync_copy(acc_sh.at[pl.ds(r0, DR), :], zbuf)
            pltpu.sync_copy(zbuf,
                            out_hbm.at[pl.ds(r0, DR), pl.ds(cid * 128, 128)])


def _sc_scatter(h, mix, s0, s1, s2, senders, receivers):
    mesh = plsc.VectorSubcoreMesh(core_axis_name="c", subcore_axis_name="s",
                                  num_cores=NC, num_subcores=NS)
    fn = pl.kernel(
        _sc_body,
        out_type=jax.ShapeDtypeStruct((N, 4 * C), jnp.float32),
        mesh=mesh,
        scratch_types=[
            pltpu.VMEM_SHARED((N, 128), jnp.float32),
            pltpu.VMEM((DR, 128), jnp.float32),
            pltpu.VMEM((K,), jnp.int32),
            pltpu.VMEM((K,), jnp.int32),
            pltpu.VMEM((K, C), jnp.float32),
            pltpu.VMEM((K, 2 * C), jnp.float32),
            pltpu.VMEM((K,), jnp.float32),
            pltpu.VMEM((K,), jnp.float32),
            pltpu.VMEM((K,), jnp.float32),
            pltpu.VMEM((K, 128), jnp.float32),
            pltpu.SemaphoreType.DMA,
        ],
        compiler_params=pltpu.CompilerParams(use_tc_tiling_on_sc=False),
    )
    return fn(h, mix, s0, s1, s2, senders, receivers)


def _down_body(in_ref, w_ref, out_ref):
    out_ref[...] = jnp.dot(in_ref[...], w_ref[...],
                           preferred_element_type=jnp.float32)


def _down(out_pre, w_big):
    bn = 2000
    return pl.pallas_call(
        _down_body,
        grid=(N // bn,),
        in_specs=[
            pl.BlockSpec((bn, 4 * C), lambda i: (i, 0)),
            pl.BlockSpec((4 * C, 4 * C), lambda i: (0, 0)),
        ],
        out_specs=pl.BlockSpec((bn, 4 * C), lambda i: (i, 0)),
        out_shape=jax.ShapeDtypeStruct((N, 4 * C), jnp.float32),
    )(out_pre, w_big)


def kernel(vectors, node_feats, radial_embedding, senders, receivers,
           W_up, M1, M2, M3, M4, Wd0, Wd1):
    inv_sqrt_c = 1.0 / math.sqrt(float(C))

    h = _compute_h(node_feats, W_up * inv_sqrt_c)

    vx = vectors[:, 0]
    vy = vectors[:, 1]
    vz = vectors[:, 2]
    mix, s0, s1, s2 = _edge_prep(
        radial_embedding, vx, vy, vz,
        M1 * (1.0 / math.sqrt(float(RAD))), M2 * 0.125, M3 * 0.125,
        M4 * 0.125)

    out_pre = _sc_scatter(h, mix, s0, s1, s2, senders, receivers)

    scale = inv_sqrt_c / math.sqrt(AVG_NEIGH)
    w_big = jnp.zeros((4 * C, 4 * C), jnp.float32)
    w_big = w_big.at[:C, :C].set(Wd0 * scale)
    for m in range(3):
        w_big = w_big.at[C * (m + 1):C * (m + 2), C + m::3].set(Wd1 * scale)

    return _down(out_pre, w_big)
